# Initial kernel scaffold; baseline (speedup 1.0000x reference)
import jax
import jax.numpy as jnp
from jax import lax
from jax.experimental import pallas as pl
from jax.experimental.pallas import tpu as pltpu
from jax.experimental.pallas import tpu_sc as plsc

_N = 10000
_G = 64
_NT = 16
_FULL = 640
_LAST = _N - 15 * _FULL


def _tile_work(x_hbm, b_hbm, colbuf, bbuf, sums, cnts, base, n_rows):
    pltpu.sync_copy(x_hbm.at[pl.ds(base, n_rows), 0], colbuf.at[pl.ds(0, n_rows)])
    pltpu.sync_copy(b_hbm.at[pl.ds(base, n_rows)], bbuf.at[pl.ds(0, n_rows)])

    lane = lax.iota(jnp.int32, 16)
    nxt_lane = jnp.minimum(lane + 1, 15)
    cc = (lane + 1).astype(jnp.float32)

    def body(j, carry):
        off = j * 16
        b = bbuf[pl.ds(off, 16)]
        v = colbuf[pl.ds(off, 16)]
        bn = plsc.load_gather(bbuf, [off + nxt_lane])
        c = jnp.cumsum(v)
        m_end = (lane == 15) | (b != bn)
        m_int = (lane < 15) & (b != bn)
        plsc.addupdate_scatter(sums, [b], c, mask=m_end)
        plsc.addupdate_scatter(sums, [bn], -c, mask=m_int)
        plsc.addupdate_scatter(cnts, [b], cc, mask=m_end)
        plsc.addupdate_scatter(cnts, [bn], -cc, mask=m_int)
        return carry

    lax.fori_loop(0, n_rows // 16, body, 0)


def _body(x_hbm, b_hbm, out_hbm, colbuf, bbuf, sums, cnts, sh_s, sh_c,
          t_s, t_c, obuf):
    wid = lax.axis_index("s")
    zeros = jnp.zeros((16,), jnp.float32)
    for j in range(_G // 16):
        sums[pl.ds(j * 16, 16)] = zeros
        cnts[pl.ds(j * 16, 16)] = zeros

    @pl.when(wid < _NT - 1)
    def _():
        _tile_work(x_hbm, b_hbm, colbuf, bbuf, sums, cnts, wid * _FULL, _FULL)

    @pl.when(wid == _NT - 1)
    def _():
        _tile_work(x_hbm, b_hbm, colbuf, bbuf, sums, cnts,
                   (_NT - 1) * _FULL, _LAST)

    pltpu.sync_copy(sums, sh_s.at[pl.ds(wid * _G, _G)])
    pltpu.sync_copy(cnts, sh_c.at[pl.ds(wid * _G, _G)])
    plsc.subcore_barrier()

    @pl.when(wid == 0)
    def _():
        pltpu.sync_copy(sh_s, t_s)
        pltpu.sync_copy(sh_c, t_c)
        for j in range(_G // 16):
            acc_s = zeros
            acc_c = zeros
            for r in range(_NT):
                acc_s = acc_s + t_s[pl.ds(r * _G + j * 16, 16)]
                acc_c = acc_c + t_c[pl.ds(r * _G + j * 16, 16)]
            obuf[pl.ds(j * 16, 16)] = acc_s / acc_c
        pltpu.sync_copy(obuf, out_hbm)


@jax.jit
def _seg_mean(x, batch):
    mesh = plsc.VectorSubcoreMesh(
        core_axis_name="c", subcore_axis_name="s", num_cores=1)
    f = pl.kernel(
        _body,
        out_type=jax.ShapeDtypeStruct((_G,), jnp.float32),
        mesh=mesh,
        scratch_types=[
            pltpu.VMEM((_FULL,), jnp.float32),
            pltpu.VMEM((_FULL,), jnp.int32),
            pltpu.VMEM((_G,), jnp.float32),
            pltpu.VMEM((_G,), jnp.float32),
            pltpu.VMEM_SHARED((_NT * _G,), jnp.float32),
            pltpu.VMEM_SHARED((_NT * _G,), jnp.float32),
            pltpu.VMEM((_NT * _G,), jnp.float32),
            pltpu.VMEM((_NT * _G,), jnp.float32),
            pltpu.VMEM((_G,), jnp.float32),
        ],
    )
    return f(x, batch)


def kernel(x, edge_index, edge_attr, batch):
    out = _seg_mean(x, batch.astype(jnp.int32))
    return out[:, None]


# speedup vs baseline: 4.2573x; 4.2573x over previous
import jax
import jax.numpy as jnp
from jax import lax
from jax.experimental import pallas as pl
from jax.experimental.pallas import tpu as pltpu
from jax.experimental.pallas import tpu_sc as plsc

_N = 10000
_G = 64
_NT = 16
_FULL = 640
_LAST = _N - 15 * _FULL


_CH = 80


def _tile_work(x_hbm, b_hbm, colbuf, bbuf, idx2d, sem, sums, cnts,
               base, n_rows):
    pltpu.sync_copy(b_hbm.at[pl.ds(base, n_rows)], bbuf.at[pl.ds(0, n_rows)])

    lane = lax.iota(jnp.int32, 16)
    n_chunks = n_rows // _CH
    for c in range(n_chunks):
        for j in range(_CH // 16):
            idx2d[c, pl.ds(j * 16, 16)] = (base + c * _CH + j * 16 + lane) * 128
    copies = [
        pltpu.async_copy(x_hbm.at[idx2d.at[c]],
                         colbuf.at[pl.ds(c * _CH, _CH)], sem)
        for c in range(n_chunks)
    ]
    for d in copies:
        d.wait()
    nxt_lane = jnp.minimum(lane + 1, 15)
    cc = (lane + 1).astype(jnp.float32)

    def body(j, carry):
        off = j * 16
        b = bbuf[pl.ds(off, 16)]
        v = colbuf[pl.ds(off, 16)]
        bn = plsc.load_gather(bbuf, [off + nxt_lane])
        c = jnp.cumsum(v)
        m_end = (lane == 15) | (b != bn)
        m_int = (lane < 15) & (b != bn)
        plsc.addupdate_scatter(sums, [b], c, mask=m_end)
        plsc.addupdate_scatter(sums, [bn], -c, mask=m_int)
        plsc.addupdate_scatter(cnts, [b], cc, mask=m_end)
        plsc.addupdate_scatter(cnts, [bn], -cc, mask=m_int)
        return carry

    lax.fori_loop(0, n_rows // 16, body, 0)


def _body(x_hbm, b_hbm, out_hbm, colbuf, bbuf, idx2d, sem, sums, cnts,
          sh_s, sh_c, t_s, t_c, obuf):
    wid = lax.axis_index("s")
    zeros = jnp.zeros((16,), jnp.float32)
    for j in range(_G // 16):
        sums[pl.ds(j * 16, 16)] = zeros
        cnts[pl.ds(j * 16, 16)] = zeros

    @pl.when(wid < _NT - 1)
    def _():
        _tile_work(x_hbm, b_hbm, colbuf, bbuf, idx2d, sem, sums, cnts,
                   wid * _FULL, _FULL)

    @pl.when(wid == _NT - 1)
    def _():
        _tile_work(x_hbm, b_hbm, colbuf, bbuf, idx2d, sem, sums, cnts,
                   (_NT - 1) * _FULL, _LAST)

    pltpu.sync_copy(sums, sh_s.at[pl.ds(wid * _G, _G)])
    pltpu.sync_copy(cnts, sh_c.at[pl.ds(wid * _G, _G)])
    plsc.subcore_barrier()

    @pl.when(wid == 0)
    def _():
        pltpu.sync_copy(sh_s, t_s)
        pltpu.sync_copy(sh_c, t_c)
        for j in range(_G // 16):
            acc_s = zeros
            acc_c = zeros
            for r in range(_NT):
                acc_s = acc_s + t_s[pl.ds(r * _G + j * 16, 16)]
                acc_c = acc_c + t_c[pl.ds(r * _G + j * 16, 16)]
            obuf[pl.ds(j * 16, 16)] = acc_s / acc_c
        pltpu.sync_copy(obuf, out_hbm)


@jax.jit
def _seg_mean(x, batch):
    mesh = plsc.VectorSubcoreMesh(
        core_axis_name="c", subcore_axis_name="s", num_cores=1)
    f = pl.kernel(
        _body,
        out_type=jax.ShapeDtypeStruct((_G,), jnp.float32),
        mesh=mesh,
        compiler_params=pltpu.CompilerParams(needs_layout_passes=False),
        scratch_types=[
            pltpu.VMEM((_FULL,), jnp.float32),
            pltpu.VMEM((_FULL,), jnp.int32),
            pltpu.VMEM((_FULL // _CH, _CH), jnp.int32),
            pltpu.SemaphoreType.DMA,
            pltpu.VMEM((_G,), jnp.float32),
            pltpu.VMEM((_G,), jnp.float32),
            pltpu.VMEM_SHARED((_NT * _G,), jnp.float32),
            pltpu.VMEM_SHARED((_NT * _G,), jnp.float32),
            pltpu.VMEM((_NT * _G,), jnp.float32),
            pltpu.VMEM((_NT * _G,), jnp.float32),
            pltpu.VMEM((_G,), jnp.float32),
        ],
    )
    return f(x.reshape(-1), batch)


def kernel(x, edge_index, edge_attr, batch):
    out = _seg_mean(x, batch.astype(jnp.int32))
    return out[:, None]
